# initial kernel scaffold (unmeasured)
import jax
import jax.numpy as jnp
from jax import lax
from jax.experimental import pallas as pl
from jax.experimental.pallas import tpu as pltpu

N_DEV = 32
N_EXP = 128
EXP_PER = N_EXP // N_DEV
CAP = 204


def kernel(x, router_W, route_idx, expert_W):
    n_tok, d_in = x.shape
    _, _, d_out = expert_W.shape

    def body(x_ref, route_ref, ew_ref, out_ref,
             comm_ref, xb_ref, hist_my, hist_all,
             send_sems, recv_sems, hsend_sems, hrecv_sems, credit_sem):
        me = lax.axis_index("i")
        left = lax.rem(me - 1 + N_DEV, N_DEV)
        right = lax.rem(me + 1, N_DEV)

        route = route_ref[:, :]
        iota_e = lax.broadcasted_iota(jnp.int32, (n_tok, N_EXP), 1)
        eq_f = (route == iota_e).astype(jnp.float32)
        hist_my[:, :] = jnp.sum(eq_f, axis=0, keepdims=True).astype(jnp.int32)

        def send_hist(j, carry):
            @pl.when(j != me)
            def _():
                rdma = pltpu.make_async_remote_copy(
                    src_ref=hist_my,
                    dst_ref=hist_all.at[pl.ds(me, 1)],
                    send_sem=hsend_sems.at[j],
                    recv_sem=hrecv_sems.at[me],
                    device_id=(j,),
                    device_id_type=pl.DeviceIdType.MESH,
                )
                rdma.start()
            return carry
        lax.fori_loop(0, N_DEV, send_hist, 0)

        def wait_hist(j, carry):
            @pl.when(j != me)
            def _():
                rdma = pltpu.make_async_remote_copy(
                    src_ref=hist_my,
                    dst_ref=hist_all.at[pl.ds(j, 1)],
                    send_sem=hsend_sems.at[j],
                    recv_sem=hrecv_sems.at[j],
                    device_id=(j,),
                    device_id_type=pl.DeviceIdType.MESH,
                )
                rdma.wait_recv()
            return carry
        lax.fori_loop(0, N_DEV, wait_hist, 0)

        def wait_hist_send(j, carry):
            @pl.when(j != me)
            def _():
                rdma = pltpu.make_async_remote_copy(
                    src_ref=hist_my,
                    dst_ref=hist_all.at[pl.ds(me, 1)],
                    send_sem=hsend_sems.at[j],
                    recv_sem=hrecv_sems.at[me],
                    device_id=(j,),
                    device_id_type=pl.DeviceIdType.MESH,
                )
                rdma.wait_send()
            return carry
        lax.fori_loop(0, N_DEV, wait_hist_send, 0)

        row32 = lax.broadcasted_iota(jnp.int32, (N_DEV, N_EXP), 0)
        prefix = jnp.sum(
            jnp.where(row32 < me, hist_all[:, :], 0), axis=0, keepdims=True
        ).astype(jnp.float32)

        ti = lax.broadcasted_iota(jnp.int32, (n_tok, n_tok), 0)
        tj = lax.broadcasted_iota(jnp.int32, (n_tok, n_tok), 1)
        tril = (tj < ti).astype(jnp.float32)
        cum_excl = jnp.dot(tril, eq_f, preferred_element_type=jnp.float32)
        rank = jnp.sum(eq_f * (cum_excl + prefix), axis=1, keepdims=True)
        keep = rank < float(CAP)

        xb_ref[:, :] = x_ref[:, :].astype(jnp.bfloat16)
        comm_ref[0] = ew_ref[:, :, :].astype(jnp.bfloat16)
        out_ref[:, :] = jnp.zeros((n_tok, d_out), jnp.float32)

        def compute(src, slot):
            xb = xb_ref[:, :]
            for j in range(EXP_PER):
                e = src * EXP_PER + j
                m = jnp.logical_and(keep, route == e)
                xm = jnp.where(m, xb, jnp.bfloat16(0))
                w = comm_ref[slot, j]
                out_ref[:, :] += jnp.dot(
                    xm, w, preferred_element_type=jnp.float32
                )

        compute(me, 0)

        def hop(h, carry):
            send_slot = lax.rem(h - 1, 2)
            recv_slot = lax.rem(h, 2)
            @pl.when(h >= 2)
            def _():
                pl.semaphore_wait(credit_sem, 1)
            rdma = pltpu.make_async_remote_copy(
                src_ref=comm_ref.at[send_slot],
                dst_ref=comm_ref.at[recv_slot],
                send_sem=send_sems.at[send_slot],
                recv_sem=recv_sems.at[recv_slot],
                device_id=(right,),
                device_id_type=pl.DeviceIdType.MESH,
            )
            rdma.start()
            rdma.wait()
            @pl.when(h <= N_DEV - 2)
            def _():
                pl.semaphore_signal(
                    credit_sem, inc=1,
                    device_id=(left,),
                    device_id_type=pl.DeviceIdType.MESH,
                )
            src = lax.rem(me - h + N_DEV, N_DEV)
            compute(src, recv_slot)
            return carry
        lax.fori_loop(1, N_DEV, hop, 0)

    return pl.pallas_call(
        body,
        out_shape=jax.ShapeDtypeStruct((n_tok, d_out), jnp.float32),
        in_specs=[
            pl.BlockSpec(memory_space=pltpu.VMEM),
            pl.BlockSpec(memory_space=pltpu.VMEM),
            pl.BlockSpec(memory_space=pltpu.VMEM),
        ],
        out_specs=pl.BlockSpec(memory_space=pltpu.VMEM),
        scratch_shapes=[
            pltpu.VMEM((2, EXP_PER, d_in, d_out), jnp.bfloat16),
            pltpu.VMEM((n_tok, d_in), jnp.bfloat16),
            pltpu.VMEM((1, N_EXP), jnp.int32),
            pltpu.VMEM((N_DEV, N_EXP), jnp.int32),
            pltpu.SemaphoreType.DMA((2,)),
            pltpu.SemaphoreType.DMA((2,)),
            pltpu.SemaphoreType.DMA((N_DEV,)),
            pltpu.SemaphoreType.DMA((N_DEV,)),
            pltpu.SemaphoreType.REGULAR,
        ],
        compiler_params=pltpu.CompilerParams(collective_id=0),
    )(x, route_idx, expert_W)


# baseline (device time: 1642376 ns/iter reference)
import jax
import jax.numpy as jnp
from jax import lax
from jax.experimental import pallas as pl
from jax.experimental.pallas import tpu as pltpu

N_DEV = 32
N_EXP = 128
EXP_PER = N_EXP // N_DEV
CAP = 204


def kernel(x, router_W, route_idx, expert_W):
    n_tok, d_in = x.shape
    _, _, d_out = expert_W.shape

    def body(x_ref, route_ref, ew_ref, out_ref,
             comm_ref, xb_ref, hist_my, hist_all,
             send_sems, recv_sems, hsend_sems, hrecv_sems, credit_sem):
        me = lax.axis_index("i")
        left = lax.rem(me - 1 + N_DEV, N_DEV)
        right = lax.rem(me + 1, N_DEV)

        route = route_ref[:, :]
        iota_e = lax.broadcasted_iota(jnp.int32, (n_tok, N_EXP), 1)
        eq_f = (route == iota_e).astype(jnp.float32)
        hist_my[:, :] = jnp.sum(eq_f, axis=0, keepdims=True).astype(jnp.int32)

        def send_hist(j, carry):
            @pl.when(j != me)
            def _():
                rdma = pltpu.make_async_remote_copy(
                    src_ref=hist_my,
                    dst_ref=hist_all.at[pl.ds(me, 1)],
                    send_sem=hsend_sems.at[j],
                    recv_sem=hrecv_sems.at[me],
                    device_id=(j,),
                    device_id_type=pl.DeviceIdType.MESH,
                )
                rdma.start()
            return carry
        lax.fori_loop(0, N_DEV, send_hist, 0)

        def wait_hist(j, carry):
            @pl.when(j != me)
            def _():
                rdma = pltpu.make_async_remote_copy(
                    src_ref=hist_my,
                    dst_ref=hist_all.at[pl.ds(j, 1)],
                    send_sem=hsend_sems.at[j],
                    recv_sem=hrecv_sems.at[j],
                    device_id=(j,),
                    device_id_type=pl.DeviceIdType.MESH,
                )
                rdma.wait_recv()
            return carry
        lax.fori_loop(0, N_DEV, wait_hist, 0)

        def wait_hist_send(j, carry):
            @pl.when(j != me)
            def _():
                rdma = pltpu.make_async_remote_copy(
                    src_ref=hist_my,
                    dst_ref=hist_all.at[pl.ds(me, 1)],
                    send_sem=hsend_sems.at[j],
                    recv_sem=hrecv_sems.at[me],
                    device_id=(j,),
                    device_id_type=pl.DeviceIdType.MESH,
                )
                rdma.wait_send()
            return carry
        lax.fori_loop(0, N_DEV, wait_hist_send, 0)

        row32 = lax.broadcasted_iota(jnp.int32, (N_DEV, N_EXP), 0)
        prefix = jnp.sum(
            jnp.where(row32 < me, hist_all[:, :], 0), axis=0, keepdims=True
        ).astype(jnp.float32)

        ti = lax.broadcasted_iota(jnp.int32, (n_tok, n_tok), 0)
        tj = lax.broadcasted_iota(jnp.int32, (n_tok, n_tok), 1)
        tril = (tj < ti).astype(jnp.float32)
        cum_excl = jnp.dot(tril, eq_f, preferred_element_type=jnp.float32)
        rank = jnp.sum(eq_f * (cum_excl + prefix), axis=1, keepdims=True)
        keep = rank < float(CAP)

        xb_ref[:, :] = x_ref[:, :].astype(jnp.bfloat16)
        comm_ref[0] = ew_ref[:, :, :].astype(jnp.bfloat16)
        out_ref[:, :] = jnp.zeros((n_tok, d_out), jnp.float32)

        def compute(src, slot):
            xb = xb_ref[:, :]
            for j in range(EXP_PER):
                e = src * EXP_PER + j
                m = jnp.logical_and(keep, route == e)
                xm = jnp.where(m, xb, jnp.bfloat16(0))
                w = comm_ref[slot, j]
                out_ref[:, :] += jnp.dot(
                    xm, w, preferred_element_type=jnp.float32
                )

        compute(me, 0)

        def hop(h, carry):
            send_slot = lax.rem(h - 1, 2)
            recv_slot = lax.rem(h, 2)
            @pl.when(h >= 2)
            def _():
                pl.semaphore_wait(credit_sem, 1)
            rdma = pltpu.make_async_remote_copy(
                src_ref=comm_ref.at[send_slot],
                dst_ref=comm_ref.at[recv_slot],
                send_sem=send_sems.at[send_slot],
                recv_sem=recv_sems.at[recv_slot],
                device_id=(right,),
                device_id_type=pl.DeviceIdType.MESH,
            )
            rdma.start()
            rdma.wait()
            @pl.when(h <= N_DEV - 2)
            def _():
                pl.semaphore_signal(
                    credit_sem, inc=1,
                    device_id=(left,),
                    device_id_type=pl.DeviceIdType.MESH,
                )
            src = lax.rem(me - h + N_DEV, N_DEV)
            compute(src, recv_slot)
            return carry
        lax.fori_loop(1, N_DEV, hop, 0)

    return pl.pallas_call(
        body,
        out_shape=jax.ShapeDtypeStruct((n_tok, d_out), jnp.float32),
        in_specs=[
            pl.BlockSpec(memory_space=pltpu.VMEM),
            pl.BlockSpec(memory_space=pltpu.VMEM),
            pl.BlockSpec(memory_space=pltpu.VMEM),
        ],
        out_specs=pl.BlockSpec(memory_space=pltpu.VMEM),
        scratch_shapes=[
            pltpu.VMEM((2, EXP_PER, d_in, d_out), jnp.bfloat16),
            pltpu.VMEM((n_tok, d_in), jnp.bfloat16),
            pltpu.VMEM((1, N_EXP), jnp.int32),
            pltpu.VMEM((N_DEV, N_EXP), jnp.int32),
            pltpu.SemaphoreType.DMA((2,)),
            pltpu.SemaphoreType.DMA((2,)),
            pltpu.SemaphoreType.DMA((N_DEV,)),
            pltpu.SemaphoreType.DMA((N_DEV,)),
            pltpu.SemaphoreType.REGULAR,
        ],
    )(x, route_idx, expert_W)
